# baseline (speedup 1.0000x reference)
import jax
import jax.numpy as jnp
from jax.experimental import pallas as pl
from jax.experimental.pallas import tpu as pltpu

_BM = 200


def _fused_kernel(x_ref, w_ref, a_ref, b_ref, out_ref, s_ref):
    @pl.when(pl.program_id(0) == 0)
    def _():
        s_ref[...] = jnp.dot(
            x_ref[...], w_ref[...], preferred_element_type=jnp.float32)

    s = s_ref[...]
    out_ref[0] = jnp.dot(a_ref[...], s, preferred_element_type=jnp.float32)
    out_ref[1] = jnp.dot(b_ref[...], s, preferred_element_type=jnp.float32)


def kernel(input, adj, weight):
    n, f_in = input.shape
    f_out = weight.shape[1]
    half_blocks = (n // 2) // _BM
    out = pl.pallas_call(
        _fused_kernel,
        grid=(half_blocks,),
        in_specs=[
            pl.BlockSpec((n, f_in), lambda i: (0, 0)),
            pl.BlockSpec((f_in, f_out), lambda i: (0, 0)),
            pl.BlockSpec((_BM, n), lambda i: (i, 0)),
            pl.BlockSpec((_BM, n), lambda i: (i + half_blocks, 0)),
        ],
        out_specs=pl.BlockSpec((2, _BM, f_out), lambda i: (0, i, 0)),
        out_shape=jax.ShapeDtypeStruct((2, n // 2, f_out), jnp.float32),
        scratch_shapes=[pltpu.VMEM((n, f_out), jnp.float32)],
    )(input, weight, adj, adj)
    return out.reshape(n, f_out)
